# baseline (speedup 1.0000x reference)
import functools

import jax
import jax.numpy as jnp
from jax.experimental import pallas as pl
from jax.experimental.pallas import tpu as pltpu

EPS = 1e-5
_PARITIES = ((0, 0), (0, 1), (1, 0), (1, 1))
_ALL_TAPS = tuple((dy, dx) for dy in (0, 1, 2) for dx in (0, 1, 2))

_CP = pltpu.CompilerParams(
    dimension_semantics=("parallel", "parallel"),
    vmem_limit_bytes=48 * 1024 * 1024,
)


def _offs(p):
    return (0, 1, 2) if p == 0 else (1, 2)


def _ptaps(a, b):
    return tuple((dy, dx) for dy in _offs(a) for dx in _offs(b))


def _tap_blocks(w, a, b, cin):
    taps = _ptaps(a, b)
    wr = w.reshape(len(taps), cin, w.shape[-1])
    return {t: wr[i] for i, t in enumerate(taps)}


def _group_weight(blocks_list, taps, cin, cout_each):
    dt = next(iter(blocks_list[0].values())).dtype
    mats = []
    for t in taps:
        cols = [blocks.get(t, jnp.zeros((cin, cout_each), dt))
                for blocks in blocks_list]
        mats.append(jnp.concatenate(cols, axis=1) if len(cols) > 1 else cols[0])
    return jnp.stack(mats, axis=0)


def _w4_xsplit(w4s):
    blocks = [_tap_blocks(w4s[j], a, b, 64) for j, (a, b) in enumerate(_PARITIES)]
    zblk = jnp.zeros((64, 3), jnp.bfloat16)
    dxmap = {(0, 0): (None, 0), (0, 1): (1, 2), (1, 0): (0, 1), (1, 1): (2, None)}
    out = {}
    for d in (0, 1):
        for side in (0, 1):
            mats = []
            for dy in (0, 1, 2):
                rows = []
                for u in (0, 1):
                    dx = dxmap[(d, side)][u]
                    cols = [blocks[j][(dy, dx)][:, :3]
                            if dx is not None and (dy, dx) in blocks[j] else zblk
                            for j in range(4)]
                    m = jnp.stack(cols, axis=-1).reshape(64, 12)
                    rows.append(jnp.pad(m, ((0, 0), (0, 116))))
                mats.append(jnp.concatenate(rows, axis=0))
            out[(d, side)] = jnp.stack(mats, axis=0)
    return out


def _final_kernel(bt, *refs):
    (x_ref, stats_ref, g_ref, be_ref, bias_ref,
     wl0, wr0, wl1, wr1, o0, o1) = refs
    tot = jnp.sum(stats_ref[...], axis=0)
    gp = tot.shape[0] // 2
    ssum = jnp.sum(tot[:gp], axis=0, keepdims=True)
    ssq = jnp.sum(tot[gp:], axis=0, keepdims=True)
    c = ssum.shape[1] // 2
    ssum = ssum[:, :c] + ssum[:, c:]
    ssq = ssq[:, :c] + ssq[:, c:]
    mean = ssum / 262144.0
    var = jnp.maximum(ssq / 262144.0 - mean * mean, 0.0)
    scale = g_ref[...] * jax.lax.rsqrt(var + EPS)
    shift = be_ref[...] - mean * scale
    sp = jnp.concatenate([scale, scale], axis=1).reshape(1, 1, 1, 128)
    tp = jnp.concatenate([shift, shift], axis=1).reshape(1, 1, 1, 128)
    x = x_ref[...]
    y = jnp.maximum(x.astype(jnp.float32) * sp + tp, 0.0).astype(jnp.bfloat16)
    zrow = jnp.zeros((bt, 1, 32, 128), jnp.bfloat16)
    yp = jnp.concatenate([zrow, y, zrow], axis=1)
    zcol = jnp.zeros((bt, 66, 1, 128), jnp.bfloat16)
    yp = jnp.concatenate([zcol, yp, zcol], axis=2)
    r2 = bt * 64 * 32
    sxo = [yp[:, :, xo:xo + 32, :] for xo in (0, 1, 2)]
    sl = {}
    for dy in (0, 1, 2):
        for xo in (0, 1, 2):
            sl[(dy, xo)] = sxo[xo][:, dy:dy + 64].reshape(r2, 128)
    for d, (o_ref, wl, wr) in enumerate(((o0, wl0, wr0), (o1, wl1, wr1))):
        z = None
        for dy in (0, 1, 2):
            zz = (jnp.dot(sl[(dy, d)], wl[dy],
                          preferred_element_type=jnp.float32)
                  + jnp.dot(sl[(dy, d + 1)], wr[dy],
                            preferred_element_type=jnp.float32))
            z = zz if z is None else z + zz
        z = jnp.tanh(z + bias_ref[...])
        o_ref[...] = z[:, :12].astype(jnp.bfloat16)


def _final_call(x3, stats3, g3, beta3, bias4, wmap, bt):
    b = x3.shape[0]
    n = b // bt
    n2 = n // 2
    r2 = b * 64 * 32
    rt = bt * 64 * 32

    def bidx(i, j):
        return i * n2 + j

    in_specs = [
        pl.BlockSpec((bt, 64, 32, 128), lambda i, j: (bidx(i, j), 0, 0, 0)),
        pl.BlockSpec(stats3.shape, lambda i, j: (0, 0, 0)),
        pl.BlockSpec((1, 64), lambda i, j: (0, 0)),
        pl.BlockSpec((1, 64), lambda i, j: (0, 0)),
        pl.BlockSpec((1, 128), lambda i, j: (0, 0)),
    ] + [pl.BlockSpec((3, 128, 128), lambda i, j: (0, 0, 0))] * 4
    return pl.pallas_call(
        functools.partial(_final_kernel, bt),
        out_shape=(jax.ShapeDtypeStruct((r2, 12), jnp.bfloat16),
                   jax.ShapeDtypeStruct((r2, 12), jnp.bfloat16)),
        grid=(2, n2),
        in_specs=in_specs,
        out_specs=(pl.BlockSpec((rt, 12), lambda i, j: (bidx(i, j), 0)),
                   pl.BlockSpec((rt, 12), lambda i, j: (bidx(i, j), 0))),
        compiler_params=_CP,
    )(x3, stats3, g3, beta3, bias4,
      wmap[(0, 0)], wmap[(0, 1)], wmap[(1, 0)], wmap[(1, 1)])


def _conv_kernel(bt, h, w, cin, groups, bn, count_prev, pairs_prev, finale,
                 pair_split, *refs):
    G = len(groups)
    k = 0
    x_ref = refs[k]; k += 1
    if bn:
        stats_ref, g_ref, be_ref = refs[k], refs[k + 1], refs[k + 2]; k += 3
    if finale:
        bias_ref = refs[k]; k += 1
    w_refs = refs[k:k + G]; k += G
    n_out = G if finale else 1
    o_refs = refs[k:k + n_out]; k += n_out
    if not finale:
        so_ref = refs[k]; k += 1

    x = x_ref[...]
    if bn:
        tot = jnp.sum(stats_ref[...], axis=0)
        gp = tot.shape[0] // 2
        ssum = jnp.sum(tot[:gp], axis=0, keepdims=True)
        ssq = jnp.sum(tot[gp:], axis=0, keepdims=True)
        if pairs_prev:
            c = ssum.shape[1] // 2
            ssum = ssum[:, :c] + ssum[:, c:]
            ssq = ssq[:, :c] + ssq[:, c:]
        mean = ssum / count_prev
        var = jnp.maximum(ssq / count_prev - mean * mean, 0.0)
        scale = g_ref[...] * jax.lax.rsqrt(var + EPS)
        shift = be_ref[...] - mean * scale
        xf = (x.astype(jnp.float32) * scale.reshape(1, 1, 1, -1)
              + shift.reshape(1, 1, 1, -1))
        x = jnp.maximum(xf, 0.0).astype(jnp.bfloat16)

    zrow = jnp.zeros((bt, 1, w, cin), jnp.bfloat16)
    yp = jnp.concatenate([zrow, x, zrow], axis=1)
    zcol = jnp.zeros((bt, h + 2, 1, cin), jnp.bfloat16)
    yp = jnp.concatenate([zcol, yp, zcol], axis=2)

    r = bt * h * w
    sdx = [yp[:, :, dx:dx + w, :] for dx in (0, 1, 2)]
    sums, sqs, zcs = [], [], []
    for gi, taps in enumerate(groups):
        z = None
        for t, (dy, dx) in enumerate(taps):
            sl = sdx[dx][:, dy:dy + h].reshape(r, cin)
            zz = jnp.dot(sl, w_refs[gi][t], preferred_element_type=jnp.float32)
            z = zz if z is None else z + zz
        if finale:
            z = jnp.tanh(z + bias_ref[...])
            o_refs[gi][...] = z[:, :12]
        else:
            zc = z.astype(jnp.bfloat16)
            zcs.append(zc)
            zf = zc.astype(jnp.float32)
            sums.append(jnp.sum(zf, axis=0, keepdims=True))
            sqs.append(jnp.sum(zf * zf, axis=0, keepdims=True))
    if not finale:
        if pair_split:
            co = zcs[0].shape[1]
            even = zcs[0].reshape(bt, h, w, co)
            odd = zcs[1].reshape(bt, h, w, co)
            il = jnp.stack([even, odd], axis=2)
            o_refs[0][...] = il.reshape(bt, 2 * h, w, co)
        else:
            co = zcs[0].shape[1]
            c00, c01, c10, c11 = [p.reshape(bt, h, w, co) for p in zcs]
            even = jnp.stack([c00, c01], axis=3)
            odd = jnp.stack([c10, c11], axis=3)
            il = jnp.stack([even, odd], axis=2)
            o_refs[0][...] = il.reshape(bt, 2 * h, 2 * w, co)
        so_ref[...] = jnp.concatenate(sums + sqs, axis=0).reshape(so_ref.shape)


def _conv_layer(x, wlist, groups, couts, bt, bn_args=None, finale_bias=None,
                pair_split=False):
    b, h, w, cin = x.shape
    n = b // bt
    n2 = n // 2
    r = b * h * w
    rt = bt * h * w
    G = len(groups)

    def bidx(i, j):
        return i * n2 + j

    in_specs = [pl.BlockSpec((bt, h, w, cin),
                             lambda i, j: (bidx(i, j), 0, 0, 0))]
    args = [x]
    bn = bn_args is not None
    if bn:
        stats_p, g_p, be_p, count_p, pairs_p = bn_args
        in_specs += [
            pl.BlockSpec(stats_p.shape, lambda i, j: (0, 0, 0)),
            pl.BlockSpec(g_p.shape, lambda i, j: (0, 0)),
            pl.BlockSpec(be_p.shape, lambda i, j: (0, 0)),
        ]
        args += [stats_p, g_p, be_p]
    else:
        count_p, pairs_p = 0.0, False
    finale = finale_bias is not None
    if finale:
        in_specs.append(pl.BlockSpec((1, 128), lambda i, j: (0, 0)))
        args.append(finale_bias)
    for wg in wlist:
        in_specs.append(pl.BlockSpec(wg.shape, lambda i, j: (0, 0, 0)))
        args.append(wg)
    out_shapes, out_specs = [], []
    if finale:
        for co in couts:
            out_shapes.append(jax.ShapeDtypeStruct((r, co), jnp.float32))
            out_specs.append(pl.BlockSpec((rt, co),
                                          lambda i, j: (bidx(i, j), 0)))
    else:
        if pair_split:
            out_shapes.append(jax.ShapeDtypeStruct(
                (b, 2 * h, w, couts[0]), jnp.bfloat16))
            out_specs.append(pl.BlockSpec((bt, 2 * h, w, couts[0]),
                                          lambda i, j: (bidx(i, j), 0, 0, 0)))
        else:
            co = couts[0]
            out_shapes.append(jax.ShapeDtypeStruct(
                (b, 2 * h, 2 * w, co), jnp.bfloat16))
            out_specs.append(pl.BlockSpec((bt, 2 * h, 2 * w, co),
                                          lambda i, j: (bidx(i, j), 0, 0, 0)))
        L = couts[0]
        out_shapes.append(jax.ShapeDtypeStruct((n, 2 * G, L), jnp.float32))
        out_specs.append(pl.BlockSpec((1, 2 * G, L),
                                      lambda i, j: (bidx(i, j), 0, 0)))
    fn = functools.partial(_conv_kernel, bt, h, w, cin, groups, bn,
                           count_p, pairs_p, finale, pair_split)
    return pl.pallas_call(
        fn,
        out_shape=tuple(out_shapes),
        grid=(2, n2),
        in_specs=in_specs,
        out_specs=tuple(out_specs),
        compiler_params=_CP,
    )(*args)


def kernel(x, fc_w, fc_b,
           w1_0, w1_1, w1_2, w1_3, g1, beta1,
           w2_0, w2_1, w2_2, w2_3, g2, beta2,
           w3_0, w3_1, w3_2, w3_3, g3, beta3,
           w4_0, w4_1, w4_2, w4_3, b4):
    B = x.shape[0]
    h0 = (jnp.dot(x, fc_w) + fc_b).astype(jnp.bfloat16)
    h0 = h0.reshape(B, 512, 8, 8).transpose(0, 2, 3, 1)

    groups4 = tuple(_ptaps(a, b) for a, b in _PARITIES)

    w1 = [w.reshape(len(_ptaps(a, b)), 512, 256)
          for w, (a, b) in zip((w1_0, w1_1, w1_2, w1_3), _PARITIES)]
    x1, stats1 = _conv_layer(h0, w1, groups4, (256,) * 4, bt=16)

    w2 = [w.reshape(len(_ptaps(a, b)), 256, 128)
          for w, (a, b) in zip((w2_0, w2_1, w2_2, w2_3), _PARITIES)]
    x2, stats2 = _conv_layer(
        x1, w2, groups4, (128,) * 4, bt=16,
        bn_args=(stats1, g1.reshape(1, -1), beta1.reshape(1, -1), 16384.0, False))

    blocks3 = [_tap_blocks(w, a, b, 128)
               for w, (a, b) in zip((w3_0, w3_1, w3_2, w3_3), _PARITIES)]
    taps_a = _ALL_TAPS
    taps_b = tuple(t for t in _ALL_TAPS if t[0] != 0)
    wa = _group_weight([blocks3[0], blocks3[1]], taps_a, 128, 64)
    wb = _group_weight([blocks3[2], blocks3[3]], taps_b, 128, 64)
    x3, stats3 = _conv_layer(
        x2, [wa, wb], (taps_a, taps_b), (128, 128), bt=8,
        bn_args=(stats2, g2.reshape(1, -1), beta2.reshape(1, -1), 65536.0, False),
        pair_split=True)

    wmap = _w4_xsplit((w4_0, w4_1, w4_2, w4_3))
    bias4 = jnp.pad(jnp.repeat(b4[:3], 4), (0, 116)).reshape(1, 128)
    z0, z1 = _final_call(x3, stats3, g3.reshape(1, -1), beta3.reshape(1, -1),
                         bias4, wmap, bt=4)
    a0 = z0.reshape(B, 64, 32, 3, 2, 2)
    a1 = z1.reshape(B, 64, 32, 3, 2, 2)
    st = jnp.stack([a0, a1], axis=3)
    out = (st.transpose(0, 4, 1, 5, 2, 3, 6).reshape(B, 3, 128, 128)
           .astype(jnp.float32))
    return out


# speedup vs baseline: 1.1570x; 1.1570x over previous
import functools

import jax
import jax.numpy as jnp
from jax.experimental import pallas as pl
from jax.experimental.pallas import tpu as pltpu

EPS = 1e-5
_PARITIES = ((0, 0), (0, 1), (1, 0), (1, 1))
_ALL_TAPS = tuple((dy, dx) for dy in (0, 1, 2) for dx in (0, 1, 2))

_CP = pltpu.CompilerParams(
    dimension_semantics=("parallel", "parallel"),
    vmem_limit_bytes=48 * 1024 * 1024,
)


def _offs(p):
    return (0, 1, 2) if p == 0 else (1, 2)


def _ptaps(a, b):
    return tuple((dy, dx) for dy in _offs(a) for dx in _offs(b))


def _tap_blocks(w, a, b, cin):
    taps = _ptaps(a, b)
    wr = w.reshape(len(taps), cin, w.shape[-1])
    return {t: wr[i] for i, t in enumerate(taps)}


def _group_weight(blocks_list, taps, cin, cout_each):
    dt = next(iter(blocks_list[0].values())).dtype
    mats = []
    for t in taps:
        cols = [blocks.get(t, jnp.zeros((cin, cout_each), dt))
                for blocks in blocks_list]
        mats.append(jnp.concatenate(cols, axis=1) if len(cols) > 1 else cols[0])
    return jnp.stack(mats, axis=0)


def _w4_xsplit(w4s):
    blocks = [_tap_blocks(w4s[j], a, b, 64) for j, (a, b) in enumerate(_PARITIES)]
    zblk = jnp.zeros((64, 3), jnp.bfloat16)
    out = {}
    for xo in (0, 1, 2):
        mats = []
        for dy in (0, 1, 2):
            rows = []
            for u in (0, 1):
                halves = []
                for d in (0, 1):
                    dx = 2 * xo + u - d - 1
                    cols = [blocks[j][(dy, dx)][:, :3]
                            if 0 <= dx <= 2 and (dy, dx) in blocks[j] else zblk
                            for j in range(4)]
                    halves.append(jnp.stack(cols, axis=-1).reshape(64, 12))
                m = jnp.concatenate(halves, axis=1)
                rows.append(jnp.pad(m, ((0, 0), (0, 104))))
            mats.append(jnp.concatenate(rows, axis=0))
        out[xo] = jnp.stack(mats, axis=0)
    return out


def _final_kernel(bt, *refs):
    x_ref, stats_ref, g_ref, be_ref, bias_ref, w0, w1, w2, o_ref = refs
    tot = jnp.sum(stats_ref[...], axis=0)
    gp = tot.shape[0] // 2
    ssum = jnp.sum(tot[:gp], axis=0, keepdims=True)
    ssq = jnp.sum(tot[gp:], axis=0, keepdims=True)
    c = ssum.shape[1] // 2
    ssum = ssum[:, :c] + ssum[:, c:]
    ssq = ssq[:, :c] + ssq[:, c:]
    mean = ssum / 262144.0
    var = jnp.maximum(ssq / 262144.0 - mean * mean, 0.0)
    scale = g_ref[...] * jax.lax.rsqrt(var + EPS)
    shift = be_ref[...] - mean * scale
    sp = jnp.concatenate([scale, scale], axis=1).reshape(1, 1, 1, 128)
    tp = jnp.concatenate([shift, shift], axis=1).reshape(1, 1, 1, 128)
    x = x_ref[...]
    y = jnp.maximum(x.astype(jnp.float32) * sp + tp, 0.0).astype(jnp.bfloat16)
    zrow = jnp.zeros((bt, 1, 32, 128), jnp.bfloat16)
    yp = jnp.concatenate([zrow, y, zrow], axis=1)
    zcol = jnp.zeros((bt, 66, 1, 128), jnp.bfloat16)
    yp = jnp.concatenate([zcol, yp, zcol], axis=2)
    r2 = bt * 64 * 32
    sxo = [yp[:, :, xo:xo + 32, :] for xo in (0, 1, 2)]
    z = None
    for xo, w_ref in enumerate((w0, w1, w2)):
        for dy in (0, 1, 2):
            sl = sxo[xo][:, dy:dy + 64].reshape(r2, 128)
            zz = jnp.dot(sl, w_ref[dy], preferred_element_type=jnp.float32)
            z = zz if z is None else z + zz
    z = jnp.tanh(z + bias_ref[...])
    o_ref[...] = z[:, :24].astype(jnp.bfloat16)


def _final_call(x3, stats3, g3, beta3, bias4, wmap, bt):
    b = x3.shape[0]
    n = b // bt
    n2 = n // 2
    r2 = b * 64 * 32
    rt = bt * 64 * 32

    def bidx(i, j):
        return i * n2 + j

    in_specs = [
        pl.BlockSpec((bt, 64, 32, 128), lambda i, j: (bidx(i, j), 0, 0, 0)),
        pl.BlockSpec(stats3.shape, lambda i, j: (0, 0, 0)),
        pl.BlockSpec((1, 64), lambda i, j: (0, 0)),
        pl.BlockSpec((1, 64), lambda i, j: (0, 0)),
        pl.BlockSpec((1, 128), lambda i, j: (0, 0)),
    ] + [pl.BlockSpec((3, 128, 128), lambda i, j: (0, 0, 0))] * 3
    return pl.pallas_call(
        functools.partial(_final_kernel, bt),
        out_shape=jax.ShapeDtypeStruct((r2, 24), jnp.bfloat16),
        grid=(2, n2),
        in_specs=in_specs,
        out_specs=pl.BlockSpec((rt, 24), lambda i, j: (bidx(i, j), 0)),
        compiler_params=_CP,
    )(x3, stats3, g3, beta3, bias4, wmap[0], wmap[1], wmap[2])


def _conv_kernel(bt, h, w, cin, groups, bn, count_prev, pairs_prev, finale,
                 pair_split, *refs):
    G = len(groups)
    k = 0
    x_ref = refs[k]; k += 1
    if bn:
        stats_ref, g_ref, be_ref = refs[k], refs[k + 1], refs[k + 2]; k += 3
    if finale:
        bias_ref = refs[k]; k += 1
    w_refs = refs[k:k + G]; k += G
    n_out = G if finale else 1
    o_refs = refs[k:k + n_out]; k += n_out
    if not finale:
        so_ref = refs[k]; k += 1

    x = x_ref[...]
    if bn:
        tot = jnp.sum(stats_ref[...], axis=0)
        gp = tot.shape[0] // 2
        ssum = jnp.sum(tot[:gp], axis=0, keepdims=True)
        ssq = jnp.sum(tot[gp:], axis=0, keepdims=True)
        if pairs_prev:
            c = ssum.shape[1] // 2
            ssum = ssum[:, :c] + ssum[:, c:]
            ssq = ssq[:, :c] + ssq[:, c:]
        mean = ssum / count_prev
        var = jnp.maximum(ssq / count_prev - mean * mean, 0.0)
        scale = g_ref[...] * jax.lax.rsqrt(var + EPS)
        shift = be_ref[...] - mean * scale
        xf = (x.astype(jnp.float32) * scale.reshape(1, 1, 1, -1)
              + shift.reshape(1, 1, 1, -1))
        x = jnp.maximum(xf, 0.0).astype(jnp.bfloat16)

    zrow = jnp.zeros((bt, 1, w, cin), jnp.bfloat16)
    yp = jnp.concatenate([zrow, x, zrow], axis=1)
    zcol = jnp.zeros((bt, h + 2, 1, cin), jnp.bfloat16)
    yp = jnp.concatenate([zcol, yp, zcol], axis=2)

    r = bt * h * w
    sdx = [yp[:, :, dx:dx + w, :] for dx in (0, 1, 2)]
    sums, sqs, zcs = [], [], []
    for gi, taps in enumerate(groups):
        z = None
        for t, (dy, dx) in enumerate(taps):
            sl = sdx[dx][:, dy:dy + h].reshape(r, cin)
            zz = jnp.dot(sl, w_refs[gi][t], preferred_element_type=jnp.float32)
            z = zz if z is None else z + zz
        if finale:
            z = jnp.tanh(z + bias_ref[...])
            o_refs[gi][...] = z[:, :12]
        else:
            zc = z.astype(jnp.bfloat16)
            zcs.append(zc)
            zf = zc.astype(jnp.float32)
            sums.append(jnp.sum(zf, axis=0, keepdims=True))
            sqs.append(jnp.sum(zf * zf, axis=0, keepdims=True))
    if not finale:
        if pair_split:
            co = zcs[0].shape[1]
            even = zcs[0].reshape(bt, h, w, co)
            odd = zcs[1].reshape(bt, h, w, co)
            il = jnp.stack([even, odd], axis=2)
            o_refs[0][...] = il.reshape(bt, 2 * h, w, co)
        else:
            co = zcs[0].shape[1]
            c00, c01, c10, c11 = [p.reshape(bt, h, w, co) for p in zcs]
            even = jnp.stack([c00, c01], axis=3)
            odd = jnp.stack([c10, c11], axis=3)
            il = jnp.stack([even, odd], axis=2)
            o_refs[0][...] = il.reshape(bt, 2 * h, 2 * w, co)
        so_ref[...] = jnp.concatenate(sums + sqs, axis=0).reshape(so_ref.shape)


def _conv_layer(x, wlist, groups, couts, bt, bn_args=None, finale_bias=None,
                pair_split=False):
    b, h, w, cin = x.shape
    n = b // bt
    n2 = n // 2
    r = b * h * w
    rt = bt * h * w
    G = len(groups)

    def bidx(i, j):
        return i * n2 + j

    in_specs = [pl.BlockSpec((bt, h, w, cin),
                             lambda i, j: (bidx(i, j), 0, 0, 0))]
    args = [x]
    bn = bn_args is not None
    if bn:
        stats_p, g_p, be_p, count_p, pairs_p = bn_args
        in_specs += [
            pl.BlockSpec(stats_p.shape, lambda i, j: (0, 0, 0)),
            pl.BlockSpec(g_p.shape, lambda i, j: (0, 0)),
            pl.BlockSpec(be_p.shape, lambda i, j: (0, 0)),
        ]
        args += [stats_p, g_p, be_p]
    else:
        count_p, pairs_p = 0.0, False
    finale = finale_bias is not None
    if finale:
        in_specs.append(pl.BlockSpec((1, 128), lambda i, j: (0, 0)))
        args.append(finale_bias)
    for wg in wlist:
        in_specs.append(pl.BlockSpec(wg.shape, lambda i, j: (0, 0, 0)))
        args.append(wg)
    out_shapes, out_specs = [], []
    if finale:
        for co in couts:
            out_shapes.append(jax.ShapeDtypeStruct((r, co), jnp.float32))
            out_specs.append(pl.BlockSpec((rt, co),
                                          lambda i, j: (bidx(i, j), 0)))
    else:
        if pair_split:
            out_shapes.append(jax.ShapeDtypeStruct(
                (b, 2 * h, w, couts[0]), jnp.bfloat16))
            out_specs.append(pl.BlockSpec((bt, 2 * h, w, couts[0]),
                                          lambda i, j: (bidx(i, j), 0, 0, 0)))
        else:
            co = couts[0]
            out_shapes.append(jax.ShapeDtypeStruct(
                (b, 2 * h, 2 * w, co), jnp.bfloat16))
            out_specs.append(pl.BlockSpec((bt, 2 * h, 2 * w, co),
                                          lambda i, j: (bidx(i, j), 0, 0, 0)))
        L = couts[0]
        out_shapes.append(jax.ShapeDtypeStruct((n, 2 * G, L), jnp.float32))
        out_specs.append(pl.BlockSpec((1, 2 * G, L),
                                      lambda i, j: (bidx(i, j), 0, 0)))
    fn = functools.partial(_conv_kernel, bt, h, w, cin, groups, bn,
                           count_p, pairs_p, finale, pair_split)
    return pl.pallas_call(
        fn,
        out_shape=tuple(out_shapes),
        grid=(2, n2),
        in_specs=in_specs,
        out_specs=tuple(out_specs),
        compiler_params=_CP,
    )(*args)


def kernel(x, fc_w, fc_b,
           w1_0, w1_1, w1_2, w1_3, g1, beta1,
           w2_0, w2_1, w2_2, w2_3, g2, beta2,
           w3_0, w3_1, w3_2, w3_3, g3, beta3,
           w4_0, w4_1, w4_2, w4_3, b4):
    B = x.shape[0]
    h0 = (jnp.dot(x, fc_w) + fc_b).astype(jnp.bfloat16)
    h0 = h0.reshape(B, 512, 8, 8).transpose(0, 2, 3, 1)

    groups4 = tuple(_ptaps(a, b) for a, b in _PARITIES)

    w1 = [w.reshape(len(_ptaps(a, b)), 512, 256)
          for w, (a, b) in zip((w1_0, w1_1, w1_2, w1_3), _PARITIES)]
    x1, stats1 = _conv_layer(h0, w1, groups4, (256,) * 4, bt=8)

    w2 = [w.reshape(len(_ptaps(a, b)), 256, 128)
          for w, (a, b) in zip((w2_0, w2_1, w2_2, w2_3), _PARITIES)]
    x2, stats2 = _conv_layer(
        x1, w2, groups4, (128,) * 4, bt=8,
        bn_args=(stats1, g1.reshape(1, -1), beta1.reshape(1, -1), 16384.0, False))

    blocks3 = [_tap_blocks(w, a, b, 128)
               for w, (a, b) in zip((w3_0, w3_1, w3_2, w3_3), _PARITIES)]
    taps_a = _ALL_TAPS
    taps_b = tuple(t for t in _ALL_TAPS if t[0] != 0)
    wa = _group_weight([blocks3[0], blocks3[1]], taps_a, 128, 64)
    wb = _group_weight([blocks3[2], blocks3[3]], taps_b, 128, 64)
    x3, stats3 = _conv_layer(
        x2, [wa, wb], (taps_a, taps_b), (128, 128), bt=8,
        bn_args=(stats2, g2.reshape(1, -1), beta2.reshape(1, -1), 65536.0, False),
        pair_split=True)

    wmap = _w4_xsplit((w4_0, w4_1, w4_2, w4_3))
    rep = jnp.repeat(b4[:3], 4)
    bias4 = jnp.pad(jnp.concatenate([rep, rep]), (0, 104)).reshape(1, 128)
    z24 = _final_call(x3, stats3, g3.reshape(1, -1), beta3.reshape(1, -1),
                      bias4, wmap, bt=4)
    st = z24.reshape(B, 64, 32, 2, 3, 2, 2)
    out = (st.transpose(0, 4, 1, 5, 2, 3, 6).reshape(B, 3, 128, 128)
           .astype(jnp.float32))
    return out
